# baseline (speedup 1.0000x reference)
import functools
import math

import jax
import jax.numpy as jnp
from jax import lax
from jax.experimental import pallas as pl
from jax.experimental.pallas import tpu as pltpu
from jax.experimental.pallas import tpu_sc as plsc

NW = 32
LANES = 16
NCAND = 224
NGROUPS = NCAND // LANES


def _rsqrt_sw(x):
    i = lax.bitcast_convert_type(x, jnp.int32)
    i = jnp.int32(0x5F3759DF) - (i >> 1)
    y = lax.bitcast_convert_type(i, jnp.float32)
    for _ in range(3):
        y = y * (1.5 - 0.5 * x * y * y)
    return y


def _sqrt_sw(x):
    return x * _rsqrt_sw(x)


def _atan(t):
    t2 = t * t
    p = jnp.float32(0.0028662257)
    p = p * t2 - jnp.float32(0.0161657367)
    p = p * t2 + jnp.float32(0.0429096138)
    p = p * t2 - jnp.float32(0.0752896400)
    p = p * t2 + jnp.float32(0.1065626393)
    p = p * t2 - jnp.float32(0.1420889944)
    p = p * t2 + jnp.float32(0.1999355085)
    p = p * t2 - jnp.float32(0.3333314528)
    p = p * t2 + jnp.float32(1.0)
    return p * t


def _ball_to_cube(x, y, z, sqrt_fn, rsqrt_fn):
    eps = 1e-12
    sq_norm = x * x + y * y + z * z
    small = sq_norm < eps
    sqn_safe = jnp.where(small, 1.0, sq_norm)
    norm = sqrt_fn(sqn_safe)
    xy2 = x * x + y * y
    cap = (1.25 * z * z) > xy2
    denom_cap = norm + jnp.abs(z)
    s_cap = sqrt_fn(3.0 * norm / jnp.where(denom_cap < eps, 1.0, denom_cap))
    xy2_safe = jnp.where(xy2 < eps, 1.0, xy2)
    s_side = norm * rsqrt_fn(xy2_safe)
    x1 = jnp.where(cap, x * s_cap, x * s_side)
    y1 = jnp.where(cap, y * s_cap, y * s_side)
    z1 = jnp.where(cap, jnp.sign(z) * norm, 1.5 * z)
    x1 = jnp.where(small, 0.0, x1)
    y1 = jnp.where(small, 0.0, y1)
    z1 = jnp.where(small, 0.0, z1)
    sq_xy = x1 * x1 + y1 * y1
    small_xy = sq_xy < eps
    nxy = sqrt_fn(jnp.where(small_xy, 1.0, sq_xy))
    condx = jnp.abs(y1) <= jnp.abs(x1)
    dx_safe = jnp.where(jnp.abs(x1) < eps, 1.0, x1)
    dy_safe = jnp.where(jnp.abs(y1) < eps, 1.0, y1)
    tmp_x = jnp.sign(x1) * nxy
    tmp_y = jnp.sign(y1) * nxy
    four_over_pi = 4.0 / math.pi
    rx = jnp.clip(x1 / dy_safe, -1.0, 1.0)
    ry = jnp.clip(y1 / dx_safe, -1.0, 1.0)
    x2 = jnp.where(condx, tmp_x, tmp_y * four_over_pi * _atan(rx))
    y2 = jnp.where(condx, tmp_x * four_over_pi * _atan(ry), tmp_y)
    x2 = jnp.where(small_xy, 0.0, x2)
    y2 = jnp.where(small_xy, 0.0, y2)
    return x2, y2, z1


def _interp2(t):
    t = jnp.clip(t, 0.0, 2.0)
    i0 = jnp.minimum(t.astype(jnp.int32), 1)
    f = t - i0.astype(jnp.float32)
    return 1.0 - f, f, i0


def _make_sc_kernel(npts, ng, cin, ppw):
    ng3 = ng * ng * ng
    n_per_batch = npts // 2
    ncp = cin // 2
    mesh = plsc.VectorSubcoreMesh(core_axis_name="c", subcore_axis_name="s",
                                  num_cores=2, num_subcores=16)
    inv25 = jnp.float32(1.0 / 2.5)

    @functools.partial(
        pl.kernel,
        out_type=(jax.ShapeDtypeStruct((npts, 27 * cin), jnp.float32),
                  jax.ShapeDtypeStruct((npts,), jnp.float32)),
        mesh=mesh,
        compiler_params=pltpu.CompilerParams(needs_layout_passes=False),
        scratch_types=[
            pltpu.VMEM((ppw,), jnp.float32),
            pltpu.VMEM((ppw,), jnp.float32),
            pltpu.VMEM((ppw,), jnp.float32),
            pltpu.VMEM((NCAND,), jnp.int32),
            pltpu.VMEM((NCAND,), jnp.int32),
            pltpu.VMEM((NCAND,), jnp.int32),
            pltpu.VMEM((ng * ng * ng * 16,), jnp.int32),
            pltpu.VMEM((256,), jnp.int32),
            pltpu.VMEM((256,), jnp.float32),
            pltpu.VMEM((256,), jnp.float32),
            pltpu.VMEM((256,), jnp.float32),
            pltpu.VMEM((27 * cin,), jnp.float32),
            pltpu.VMEM((ppw,), jnp.float32),
        ],
    )
    def sc_kernel(pcx_h, pcy_h, pcz_h, tbl_h, offx_h, offy_h, offz_h,
                  a_h, cnt_h,
                  pcx_v, pcy_v, pcz_v, offx_v, offy_v, offz_v,
                  tbl_v, jb16, rxb, ryb, rzb, a_v, cnt_v):
        cid = lax.axis_index("c")
        sid = lax.axis_index("s")
        wid = sid * 2 + cid
        base = wid * ppw
        bsl = base // n_per_batch
        lanes = lax.iota(jnp.int32, LANES)

        pltpu.sync_copy(pcx_h.at[pl.ds(base, ppw)], pcx_v)
        pltpu.sync_copy(pcy_h.at[pl.ds(base, ppw)], pcy_v)
        pltpu.sync_copy(pcz_h.at[pl.ds(base, ppw)], pcz_v)
        pltpu.sync_copy(offx_h, offx_v)
        pltpu.sync_copy(offy_h, offy_v)
        pltpu.sync_copy(offz_h, offz_v)
        pltpu.sync_copy(tbl_h.at[bsl], tbl_v)

        zi = jnp.zeros((LANES,), jnp.int32)
        zf = jnp.zeros((LANES,), jnp.float32)
        for k in range(256 // LANES):
            jb16[pl.ds(k * LANES, LANES)] = zi

        def point_body(p, carry):
            pidx = jnp.full((LANES,), p, jnp.int32)
            px = plsc.load_gather(pcx_v, [pidx])
            py = plsc.load_gather(pcy_v, [pidx])
            pz = plsc.load_gather(pcz_v, [pidx])
            bx = px.astype(jnp.int32)
            by = py.astype(jnp.int32)
            bz = pz.astype(jnp.int32)

            def g_body(g, off):
                gsl = pl.ds(g * LANES, LANES)
                ox = offx_v[gsl]
                oy = offy_v[gsl]
                oz = offz_v[gsl]
                cx = bx + ox
                cy = by + oy
                cz = bz + oz
                valid = ((cx >= 0) & (cx < ng) & (cy >= 0) & (cy < ng)
                         & (cz >= 0) & (cz < ng))
                rx = (cx.astype(jnp.float32) - px) * inv25
                ry = (cy.astype(jnp.float32) - py) * inv25
                rz = (cz.astype(jnp.float32) - pz) * inv25
                dist2 = rx * rx + ry * ry + rz * rz
                m = (dist2 <= 1.0) & valid
                cum = plsc.cumsum(m.astype(jnp.int32))
                dest = off + cum - 1
                jloc = (cx * (ng * ng) + cy * ng + cz) * ncp
                plsc.store_scatter(jb16, [dest], jloc, mask=m)
                plsc.store_scatter(rxb, [dest], rx, mask=m)
                plsc.store_scatter(ryb, [dest], ry, mask=m)
                plsc.store_scatter(rzb, [dest], rz, mask=m)
                return off + plsc.all_reduce_population_count(m)

            off = lax.fori_loop(0, NGROUPS, g_body, zi)
            s_total = jnp.max(off)

            for k in range(27 * cin // LANES):
                a_v[pl.ds(k * LANES, LANES)] = zf

            n_sgroups = (s_total + (LANES - 1)) >> 4

            @plsc.parallel_loop(0, n_sgroups * LANES, LANES)
            def s_body(soff):
                ssl = pl.ds(soff, LANES)
                rx = rxb[ssl]
                ry = ryb[ssl]
                rz = rzb[ssl]
                jb = jb16[ssl]
                live = (soff + lanes) < s_total
                u, v, w = _ball_to_cube(rx, ry, rz, _sqrt_sw, _rsqrt_sw)
                xa0, xa1, ix = _interp2(u + 1.0)
                ya0, ya1, iy = _interp2(v + 1.0)
                za0, za1, iz = _interp2(w + 1.0)
                za0 = jnp.where(live, za0, 0.0)
                za1 = jnp.where(live, za1, 0.0)
                t0 = iz * 9 + iy * 3 + ix
                zy00 = za0 * ya0
                zy01 = za0 * ya1
                zy10 = za1 * ya0
                zy11 = za1 * ya1
                w8 = (zy00 * xa0, zy00 * xa1, zy01 * xa0, zy01 * xa1,
                      zy10 * xa0, zy10 * xa1, zy11 * xa0, zy11 * xa1)
                dts = (0, 1, 3, 4, 9, 10, 12, 13)
                tbk = [(t0 + dts[k]) * cin for k in range(8)]
                for cp in range(ncp):
                    pk = plsc.load_gather(tbl_v, [jb + cp])
                    pb = plsc.bitcast(pk, jnp.bfloat16)
                    lo, hi = plsc.unpack(pb, format=plsc.PackFormat.INTERLEAVED)
                    for k in range(8):
                        plsc.addupdate_scatter(a_v, [tbk[k] + cp],
                                               w8[k] * lo)
                        plsc.addupdate_scatter(a_v, [tbk[k] + (cp + ncp)],
                                               w8[k] * hi)

            pltpu.sync_copy(a_v, a_h.at[base + p])
            plsc.store_scatter(cnt_v, [pidx],
                               jnp.full((LANES,), s_total.astype(jnp.float32)),
                               mask=lanes == 0)
            return carry

        lax.fori_loop(0, ppw, point_body, 0)
        pltpu.sync_copy(cnt_v, cnt_h.at[pl.ds(base, ppw)])

    return sc_kernel


def _fin_body(a_ref, k_ref, c_ref, o_ref):
    acc = lax.dot_general(a_ref[...], k_ref[...], (((1,), (0,)), ((), ())),
                          preferred_element_type=jnp.float32)
    o_ref[...] = acc / jnp.maximum(c_ref[...], 1.0)


def kernel(input, pos, grid_pos, dx, kernel):
    Bb, cin = input.shape[0], input.shape[1]
    ng = input.shape[2]
    n = pos.shape[1]
    cout = kernel.shape[-1]
    npts = Bb * n
    ppw = npts // NW
    ng3 = ng * ng * ng
    ncp = cin // 2

    pc = (pos * (1.0 / dx)).reshape(npts, 3)
    pcx = pc[:, 0]
    pcy = pc[:, 1]
    pcz = pc[:, 2]

    feat2 = jnp.transpose(input, (0, 2, 3, 4, 1)).reshape(Bb, ng3, cin)
    fb = feat2.astype(jnp.bfloat16)
    bits = lax.bitcast_convert_type(fb, jnp.uint16).astype(jnp.uint32)
    packed = bits[..., :ncp] | (bits[..., ncp:] << 16)
    tbl = lax.bitcast_convert_type(packed, jnp.int32).reshape(Bb, ng3 * ncp)

    import numpy as np
    ids = np.arange(NCAND)
    offx = np.where(ids < 216, ids // 36 - 2, 1000).astype(np.int32)
    offy = np.where(ids < 216, (ids // 6) % 6 - 2, 1000).astype(np.int32)
    offz = np.where(ids < 216, ids % 6 - 2, 1000).astype(np.int32)

    sc = _make_sc_kernel(npts, ng, cin, ppw)
    a_flat, cnt = sc(pcx, pcy, pcz, tbl,
                     jnp.asarray(offx), jnp.asarray(offy), jnp.asarray(offz))

    kflat = kernel.reshape(27, cin, cout)
    kflat = jnp.concatenate([kflat[:, :ncp, :], kflat[:, ncp:, :]], axis=1)
    kflat = kflat.reshape(27 * cin, cout)
    cnt2 = cnt.reshape(npts, 1)
    n_tile = 256
    out2 = pl.pallas_call(
        _fin_body,
        grid=(npts // n_tile,),
        in_specs=[
            pl.BlockSpec((n_tile, 27 * cin), lambda i: (i, 0)),
            pl.BlockSpec((27 * cin, cout), lambda i: (0, 0)),
            pl.BlockSpec((n_tile, 1), lambda i: (i, 0)),
        ],
        out_specs=pl.BlockSpec((n_tile, cout), lambda i: (i, 0)),
        out_shape=jax.ShapeDtypeStruct((npts, cout), jnp.float32),
    )(a_flat, kflat, cnt2)
    return out2.reshape(Bb, n, cout)


# speedup vs baseline: 2.9891x; 2.9891x over previous
import functools
import math

import jax
import jax.numpy as jnp
from jax import lax
from jax.experimental import pallas as pl
from jax.experimental.pallas import tpu as pltpu
from jax.experimental.pallas import tpu_sc as plsc

NW = 32
LANES = 16
NCAND = 224
NGROUPS = NCAND // LANES


def _rsqrt_sw(x):
    i = lax.bitcast_convert_type(x, jnp.int32)
    i = jnp.int32(0x5F3759DF) - (i >> 1)
    y = lax.bitcast_convert_type(i, jnp.float32)
    for _ in range(3):
        y = y * (1.5 - 0.5 * x * y * y)
    return y


def _sqrt_sw(x):
    return x * _rsqrt_sw(x)


def _atan(t):
    t2 = t * t
    p = jnp.float32(0.0028662257)
    p = p * t2 - jnp.float32(0.0161657367)
    p = p * t2 + jnp.float32(0.0429096138)
    p = p * t2 - jnp.float32(0.0752896400)
    p = p * t2 + jnp.float32(0.1065626393)
    p = p * t2 - jnp.float32(0.1420889944)
    p = p * t2 + jnp.float32(0.1999355085)
    p = p * t2 - jnp.float32(0.3333314528)
    p = p * t2 + jnp.float32(1.0)
    return p * t


def _ball_to_cube(x, y, z, sqrt_fn, rsqrt_fn):
    eps = 1e-12
    sq_norm = x * x + y * y + z * z
    small = sq_norm < eps
    sqn_safe = jnp.where(small, 1.0, sq_norm)
    norm = sqrt_fn(sqn_safe)
    xy2 = x * x + y * y
    cap = (1.25 * z * z) > xy2
    denom_cap = norm + jnp.abs(z)
    s_cap = sqrt_fn(3.0 * norm / jnp.where(denom_cap < eps, 1.0, denom_cap))
    xy2_safe = jnp.where(xy2 < eps, 1.0, xy2)
    s_side = norm * rsqrt_fn(xy2_safe)
    x1 = jnp.where(cap, x * s_cap, x * s_side)
    y1 = jnp.where(cap, y * s_cap, y * s_side)
    z1 = jnp.where(cap, jnp.sign(z) * norm, 1.5 * z)
    x1 = jnp.where(small, 0.0, x1)
    y1 = jnp.where(small, 0.0, y1)
    z1 = jnp.where(small, 0.0, z1)
    sq_xy = x1 * x1 + y1 * y1
    small_xy = sq_xy < eps
    nxy = sqrt_fn(jnp.where(small_xy, 1.0, sq_xy))
    condx = jnp.abs(y1) <= jnp.abs(x1)
    dx_safe = jnp.where(jnp.abs(x1) < eps, 1.0, x1)
    dy_safe = jnp.where(jnp.abs(y1) < eps, 1.0, y1)
    tmp_x = jnp.sign(x1) * nxy
    tmp_y = jnp.sign(y1) * nxy
    four_over_pi = 4.0 / math.pi
    rx = jnp.clip(x1 / dy_safe, -1.0, 1.0)
    ry = jnp.clip(y1 / dx_safe, -1.0, 1.0)
    x2 = jnp.where(condx, tmp_x, tmp_y * four_over_pi * _atan(rx))
    y2 = jnp.where(condx, tmp_x * four_over_pi * _atan(ry), tmp_y)
    x2 = jnp.where(small_xy, 0.0, x2)
    y2 = jnp.where(small_xy, 0.0, y2)
    return x2, y2, z1


def _interp2(t):
    t = jnp.clip(t, 0.0, 2.0)
    i0 = jnp.minimum(t.astype(jnp.int32), 1)
    f = t - i0.astype(jnp.float32)
    return 1.0 - f, f, i0


def _make_sc_kernel(npts, ng, cin, ppw):
    ng3 = ng * ng * ng
    n_per_batch = npts // 2
    ncp = cin // 2
    mesh = plsc.VectorSubcoreMesh(core_axis_name="c", subcore_axis_name="s",
                                  num_cores=2, num_subcores=16)
    inv25 = jnp.float32(1.0 / 2.5)

    @functools.partial(
        pl.kernel,
        out_type=(jax.ShapeDtypeStruct((npts, 27 * cin), jnp.float32),
                  jax.ShapeDtypeStruct((npts,), jnp.float32)),
        mesh=mesh,
        compiler_params=pltpu.CompilerParams(needs_layout_passes=False),
        scratch_types=[
            pltpu.VMEM((ppw,), jnp.float32),
            pltpu.VMEM((ppw,), jnp.float32),
            pltpu.VMEM((ppw,), jnp.float32),
            pltpu.VMEM((NCAND,), jnp.int32),
            pltpu.VMEM((NCAND,), jnp.int32),
            pltpu.VMEM((NCAND,), jnp.int32),
            pltpu.VMEM((ng * ng * ng * 16,), jnp.int32),
            pltpu.VMEM((256,), jnp.int32),
            pltpu.VMEM((256,), jnp.float32),
            pltpu.VMEM((256,), jnp.float32),
            pltpu.VMEM((256,), jnp.float32),
            pltpu.VMEM((27 * cin,), jnp.float32),
            pltpu.VMEM((ppw,), jnp.float32),
        ],
    )
    def sc_kernel(pcx_h, pcy_h, pcz_h, tbl_h, offx_h, offy_h, offz_h,
                  a_h, cnt_h,
                  pcx_v, pcy_v, pcz_v, offx_v, offy_v, offz_v,
                  tbl_v, jb16, rxb, ryb, rzb, a_v, cnt_v):
        cid = lax.axis_index("c")
        sid = lax.axis_index("s")
        wid = sid * 2 + cid
        base = wid * ppw
        bsl = base // n_per_batch
        lanes = lax.iota(jnp.int32, LANES)

        pltpu.sync_copy(pcx_h.at[pl.ds(base, ppw)], pcx_v)
        pltpu.sync_copy(pcy_h.at[pl.ds(base, ppw)], pcy_v)
        pltpu.sync_copy(pcz_h.at[pl.ds(base, ppw)], pcz_v)
        pltpu.sync_copy(offx_h, offx_v)
        pltpu.sync_copy(offy_h, offy_v)
        pltpu.sync_copy(offz_h, offz_v)
        pltpu.sync_copy(tbl_h.at[bsl], tbl_v)

        zi = jnp.zeros((LANES,), jnp.int32)
        zf = jnp.zeros((LANES,), jnp.float32)
        for k in range(256 // LANES):
            jb16[pl.ds(k * LANES, LANES)] = zi

        def point_body(p, carry):
            pidx = jnp.full((LANES,), p, jnp.int32)
            px = plsc.load_gather(pcx_v, [pidx])
            py = plsc.load_gather(pcy_v, [pidx])
            pz = plsc.load_gather(pcz_v, [pidx])
            bx = px.astype(jnp.int32)
            by = py.astype(jnp.int32)
            bz = pz.astype(jnp.int32)

            def g_body(g, off):
                gsl = pl.ds(g * LANES, LANES)
                ox = offx_v[gsl]
                oy = offy_v[gsl]
                oz = offz_v[gsl]
                cx = bx + ox
                cy = by + oy
                cz = bz + oz
                valid = ((cx >= 0) & (cx < ng) & (cy >= 0) & (cy < ng)
                         & (cz >= 0) & (cz < ng))
                rx = (cx.astype(jnp.float32) - px) * inv25
                ry = (cy.astype(jnp.float32) - py) * inv25
                rz = (cz.astype(jnp.float32) - pz) * inv25
                dist2 = rx * rx + ry * ry + rz * rz
                m = (dist2 <= 1.0) & valid
                cum = plsc.cumsum(m.astype(jnp.int32))
                dest = off + cum - 1
                jloc = (cx * (ng * ng) + cy * ng + cz) * ncp
                plsc.store_scatter(jb16, [dest], jloc, mask=m)
                plsc.store_scatter(rxb, [dest], rx, mask=m)
                plsc.store_scatter(ryb, [dest], ry, mask=m)
                plsc.store_scatter(rzb, [dest], rz, mask=m)
                return off + plsc.all_reduce_population_count(m)

            off = lax.fori_loop(0, NGROUPS, g_body, zi)
            s_total = jnp.max(off)

            for k in range(27 * cin // LANES):
                a_v[pl.ds(k * LANES, LANES)] = zf

            n_sgroups = (s_total + (LANES - 1)) >> 4

            @plsc.parallel_loop(0, n_sgroups * LANES, LANES)
            def s_body(soff):
                ssl = pl.ds(soff, LANES)
                rx = rxb[ssl]
                ry = ryb[ssl]
                rz = rzb[ssl]
                jb = jb16[ssl]
                live = (soff + lanes) < s_total
                u, v, w = _ball_to_cube(rx, ry, rz, _sqrt_sw, _rsqrt_sw)
                xa0, xa1, ix = _interp2(u + 1.0)
                ya0, ya1, iy = _interp2(v + 1.0)
                za0, za1, iz = _interp2(w + 1.0)
                za0 = jnp.where(live, za0, 0.0)
                za1 = jnp.where(live, za1, 0.0)
                t0 = iz * 9 + iy * 3 + ix
                zy00 = za0 * ya0
                zy01 = za0 * ya1
                zy10 = za1 * ya0
                zy11 = za1 * ya1
                w8 = (zy00 * xa0, zy00 * xa1, zy01 * xa0, zy01 * xa1,
                      zy10 * xa0, zy10 * xa1, zy11 * xa0, zy11 * xa1)
                dts = (0, 1, 3, 4, 9, 10, 12, 13)
                tbk = [(t0 + dts[k]) * cin for k in range(8)]
                for cp in range(ncp):
                    pk = plsc.load_gather(tbl_v, [jb + cp])
                    pb = plsc.bitcast(pk, jnp.bfloat16)
                    lo, hi = plsc.unpack(pb, format=plsc.PackFormat.INTERLEAVED)
                    for k in range(8):
                        plsc.addupdate_scatter(a_v, [tbk[k] + cp],
                                               w8[k] * lo)
                        plsc.addupdate_scatter(a_v, [tbk[k] + (cp + ncp)],
                                               w8[k] * hi)

            pltpu.sync_copy(a_v, a_h.at[base + p])
            plsc.store_scatter(cnt_v, [pidx],
                               jnp.full((LANES,), s_total.astype(jnp.float32)),
                               mask=lanes == 0)
            return carry

        lax.fori_loop(0, ppw, point_body, 0)
        pltpu.sync_copy(cnt_v, cnt_h.at[pl.ds(base, ppw)])

    return sc_kernel


def _fin_body(a_ref, k_ref, c_ref, o_ref):
    acc = lax.dot_general(a_ref[...], k_ref[...], (((1,), (0,)), ((), ())),
                          preferred_element_type=jnp.float32)
    o_ref[...] = acc / jnp.maximum(c_ref[...], 1.0)


def _dense_body(px_ref, py_ref, pz_ref, gx_ref, gy_ref, gz_ref, feat_ref,
                kflat_ref, out_ref, *, n_tile, g_chunk, n_gchunks, cin):
    px = px_ref[0]
    py = py_ref[0]
    pz = pz_ref[0]

    def chunk(c, carry):
        acc, cnt = carry
        gsl = pl.ds(c * g_chunk, g_chunk)
        gx = gx_ref[0, :, gsl]
        gy = gy_ref[0, :, gsl]
        gz = gz_ref[0, :, gsl]
        featc = feat_ref[0, gsl, :]
        rx = gx - px
        ry = gy - py
        rz = gz - pz
        dist2 = rx * rx + ry * ry + rz * rz
        mask = (dist2 <= 1.0).astype(jnp.float32)
        u, v, w = _ball_to_cube(rx, ry, rz, jnp.sqrt,
                                lambda t: 1.0 / jnp.sqrt(t))
        xa0, xa1, _ix = _interp2(u + 1.0)
        ya0, ya1, _iy = _interp2(v + 1.0)
        za0, za1, _iz = _interp2(w + 1.0)
        wx = (jnp.where(_ix == 0, xa0, 0.0),
              jnp.where(_ix == 0, xa1, xa0),
              jnp.where(_ix == 0, 0.0, xa1))
        wy = (jnp.where(_iy == 0, ya0, 0.0),
              jnp.where(_iy == 0, ya1, ya0),
              jnp.where(_iy == 0, 0.0, ya1))
        wz = (jnp.where(_iz == 0, za0, 0.0),
              jnp.where(_iz == 0, za1, za0),
              jnp.where(_iz == 0, 0.0, za1))
        parts = []
        for kz in range(3):
            for ky in range(3):
                wzy = wz[kz] * wy[ky] * mask
                for kx in range(3):
                    wk = wzy * wx[kx]
                    parts.append(
                        lax.dot_general(
                            wk, featc, (((1,), (0,)), ((), ())),
                            preferred_element_type=jnp.float32))
        acc = acc + jnp.concatenate(parts, axis=1)
        cnt = cnt + jnp.sum(mask, axis=1, keepdims=True)
        return acc, cnt

    acc0 = jnp.zeros((n_tile, 27 * cin), jnp.float32)
    cnt0 = jnp.zeros((n_tile, 1), jnp.float32)
    acc, cnt = lax.fori_loop(0, n_gchunks, chunk, (acc0, cnt0))
    out = lax.dot_general(acc, kflat_ref[...], (((1,), (0,)), ((), ())),
                          preferred_element_type=jnp.float32)
    out_ref[0] = out / jnp.maximum(cnt, 1.0)


def _dense_part(pos_tc, grid_pos, inv_r, grid_feat, kflat, n_tile=256,
                g_chunk=1024):
    Bb, n_tc = pos_tc.shape[0], pos_tc.shape[1]
    g, cin = grid_feat.shape[1], grid_feat.shape[2]
    cout = kflat.shape[-1]
    ps = pos_tc * inv_r
    px = ps[:, :, 0:1]
    py = ps[:, :, 1:2]
    pz = ps[:, :, 2:3]
    gs = (grid_pos * inv_r).T.reshape(1, 3, g)
    gx = gs[:, 0:1, :]
    gy = gs[:, 1:2, :]
    gz = gs[:, 2:3, :]
    body = functools.partial(_dense_body, n_tile=n_tile, g_chunk=g_chunk,
                             n_gchunks=g // g_chunk, cin=cin)
    return pl.pallas_call(
        body,
        grid=(Bb, n_tc // n_tile),
        in_specs=[
            pl.BlockSpec((1, n_tile, 1), lambda b, i: (b, i, 0)),
            pl.BlockSpec((1, n_tile, 1), lambda b, i: (b, i, 0)),
            pl.BlockSpec((1, n_tile, 1), lambda b, i: (b, i, 0)),
            pl.BlockSpec((1, 1, g), lambda b, i: (0, 0, 0)),
            pl.BlockSpec((1, 1, g), lambda b, i: (0, 0, 0)),
            pl.BlockSpec((1, 1, g), lambda b, i: (0, 0, 0)),
            pl.BlockSpec((1, g, cin), lambda b, i: (b, 0, 0)),
            pl.BlockSpec((27 * cin, cout), lambda b, i: (0, 0)),
        ],
        out_specs=pl.BlockSpec((1, n_tile, cout), lambda b, i: (b, i, 0)),
        out_shape=jax.ShapeDtypeStruct((Bb, n_tc, cout), jnp.float32),
    )(px, py, pz, gx, gy, gz, grid_feat, kflat)


SC_Q = 256


def kernel(input, pos, grid_pos, dx, kernel):
    Bb, cin = input.shape[0], input.shape[1]
    ng = input.shape[2]
    n = pos.shape[1]
    cout = kernel.shape[-1]
    q = SC_Q
    npts = Bb * q
    ppw = npts // NW
    ng3 = ng * ng * ng
    ncp = cin // 2

    pc = (pos[:, :q, :] * (1.0 / dx)).reshape(npts, 3)
    pcx = pc[:, 0]
    pcy = pc[:, 1]
    pcz = pc[:, 2]

    feat2 = jnp.transpose(input, (0, 2, 3, 4, 1)).reshape(Bb, ng3, cin)
    fb = feat2.astype(jnp.bfloat16)
    bits = lax.bitcast_convert_type(fb, jnp.uint16).astype(jnp.uint32)
    packed = bits[..., :ncp] | (bits[..., ncp:] << 16)
    tbl = lax.bitcast_convert_type(packed, jnp.int32).reshape(Bb, ng3 * ncp)

    import numpy as np
    ids = np.arange(NCAND)
    offx = np.where(ids < 216, ids // 36 - 2, 1000).astype(np.int32)
    offy = np.where(ids < 216, (ids // 6) % 6 - 2, 1000).astype(np.int32)
    offz = np.where(ids < 216, ids % 6 - 2, 1000).astype(np.int32)

    sc = _make_sc_kernel(npts, ng, cin, ppw)
    a_flat, cnt = sc(pcx, pcy, pcz, tbl,
                     jnp.asarray(offx), jnp.asarray(offy), jnp.asarray(offz))

    kflat = kernel.reshape(27 * cin, cout)

    grid_feat = feat2
    out_tc = _dense_part(pos[:, q:, :], grid_pos, 1.0 / (dx * 2.5),
                         grid_feat, kflat)

    cnt2 = cnt.reshape(npts, 1)
    n_tile = min(256, npts)
    out_sc = pl.pallas_call(
        _fin_body,
        grid=(npts // n_tile,),
        in_specs=[
            pl.BlockSpec((n_tile, 27 * cin), lambda i: (i, 0)),
            pl.BlockSpec((27 * cin, cout), lambda i: (0, 0)),
            pl.BlockSpec((n_tile, 1), lambda i: (i, 0)),
        ],
        out_specs=pl.BlockSpec((n_tile, cout), lambda i: (i, 0)),
        out_shape=jax.ShapeDtypeStruct((npts, cout), jnp.float32),
    )(a_flat, kflat, cnt2)
    return jnp.concatenate([out_sc.reshape(Bb, q, cout), out_tc], axis=1)
